# baseline (speedup 1.0000x reference)
import functools

import jax
import jax.numpy as jnp
from jax import lax
from jax.experimental import pallas as pl
from jax.experimental.pallas import tpu as pltpu
from jax.experimental.pallas import tpu_sc as plsc

L = 3
D = 768
E = 8
F = 768
NH = 12
HD = 64
TB = 512


def _ln(h, g, b):
    mu = jnp.mean(h, axis=-1, keepdims=True)
    var = jnp.mean((h - mu) ** 2, axis=-1, keepdims=True)
    return (h - mu) / jnp.sqrt(var + 1e-5) * g + b


def _embed_k(x_ref, wp_ref, bp_ref, o_ref):
    o_ref[...] = x_ref[...] * wp_ref[...] + bp_ref[...]


def _qkv_k(h_ref, g_ref, b_ref, wq_ref, bq_ref, wk_ref, bk_ref, wv_ref,
           bv_ref, q_ref, k_ref, v_ref):
    a = _ln(h_ref[...], g_ref[...], b_ref[...])
    q_ref[...] = jnp.dot(a, wq_ref[...], preferred_element_type=jnp.float32) + bq_ref[...]
    k_ref[...] = jnp.dot(a, wk_ref[...], preferred_element_type=jnp.float32) + bk_ref[...]
    v_ref[...] = jnp.dot(a, wv_ref[...], preferred_element_type=jnp.float32) + bv_ref[...]


def _attn_k(q_ref, k_ref, v_ref, o_ref):
    for hh in range(2):
        q = q_ref[:, hh * HD:(hh + 1) * HD] * (1.0 / 8.0)
        k = k_ref[:, hh * HD:(hh + 1) * HD]
        v = v_ref[:, hh * HD:(hh + 1) * HD]
        s = jax.lax.dot_general(q, k, (((1,), (1,)), ((), ())),
                                preferred_element_type=jnp.float32)
        p = jnp.exp(s)
        o = jnp.dot(p, v, preferred_element_type=jnp.float32)
        o_ref[:, hh * HD:(hh + 1) * HD] = o / jnp.sum(p, axis=-1, keepdims=True)


def _oproj_k(h_ref, o_ref, wo_ref, bo_ref, out_ref):
    out_ref[...] = h_ref[...] + jnp.dot(
        o_ref[...], wo_ref[...], preferred_element_type=jnp.float32) + bo_ref[...]


BLK = 256


def _route_k(h_ref, g_ref, b_ref, gw_ref, gb_ref, m_ref, idx_ref, gate_ref,
             imp_ref, load_ref, loss_ref, *, nblocks):
    i = pl.program_id(0)
    h = h_ref[...]
    m = _ln(h, g_ref[...], b_ref[...])
    m_ref[...] = m

    logits = jnp.dot(m, gw_ref[...], preferred_element_type=jnp.float32) + gb_ref[...]
    logits = logits - jnp.max(logits, axis=-1, keepdims=True)
    ex = jnp.exp(logits)
    probs = ex / jnp.sum(ex, axis=-1, keepdims=True)

    eio = jax.lax.broadcasted_iota(jnp.int32, probs.shape, 1)
    m1 = jnp.max(probs, axis=-1, keepdims=True)
    idx1 = jnp.min(jnp.where(probs == m1, eio, E), axis=-1, keepdims=True)
    masked = jnp.where(eio == idx1, -1.0, probs)
    m2 = jnp.max(masked, axis=-1, keepdims=True)
    idx2 = jnp.min(jnp.where(masked == m2, eio, E), axis=-1, keepdims=True)
    gsum = m1 + m2
    idx_ref[...] = jnp.concatenate([idx1, idx2], axis=1)
    gate_ref[...] = jnp.concatenate([m1 / gsum, m2 / gsum], axis=1)

    imp_blk = jnp.sum(probs, axis=0, keepdims=True)
    load_blk = (jnp.sum(jnp.where(eio == idx1, 1.0, 0.0), axis=0, keepdims=True)
                + jnp.sum(jnp.where(eio == idx2, 1.0, 0.0), axis=0, keepdims=True))

    @pl.when(i == 0)
    def _():
        imp_ref[...] = jnp.zeros_like(imp_ref)
        load_ref[...] = jnp.zeros_like(load_ref)

    imp_ref[...] += imp_blk
    load_ref[...] += load_blk

    @pl.when(i == nblocks - 1)
    def _():
        n_tok = nblocks * h.shape[0]
        loss_ref[...] = ((E / (n_tok * n_tok)) *
                         jnp.sum(imp_ref[...] * load_ref[...],
                                 axis=(0, 1), keepdims=True))


def _rank_k(idx_ref, dest_ref, bmap_ref, rank_scr, *, n_tok, nblk):
    nch = n_tok // BLK
    r_io = jax.lax.broadcasted_iota(jnp.int32, (BLK, BLK), 0)
    c_io = jax.lax.broadcasted_iota(jnp.int32, (BLK, BLK), 1)
    ltri = (r_io > c_io).astype(jnp.float32)
    eio = jax.lax.broadcasted_iota(jnp.int32, (BLK, E), 1)

    carry = jnp.zeros((1, E), jnp.float32)
    for k in range(2):
        for c in range(nch):
            t0 = c * BLK
            ohc = (idx_ref[pl.ds(t0, BLK), k:k + 1] == eio).astype(jnp.float32)
            rank_full = jnp.dot(ltri, ohc,
                                preferred_element_type=jnp.float32) + carry
            rank_scr[pl.ds(t0, BLK), k:k + 1] = jnp.sum(
                ohc * rank_full, axis=1, keepdims=True)
            carry = carry + jnp.sum(ohc, axis=0, keepdims=True)

    counts = carry
    padded = jnp.ceil(counts * (1.0 / BLK)) * BLK
    f_io = jax.lax.broadcasted_iota(jnp.int32, (E, E), 0)
    e_io = jax.lax.broadcasted_iota(jnp.int32, (E, E), 1)
    ustri = (f_io < e_io).astype(jnp.float32)
    offs = jnp.dot(padded, ustri, preferred_element_type=jnp.float32)
    ends = offs + padded

    for k in range(2):
        for c in range(nch):
            t0 = c * BLK
            ohc = (idx_ref[pl.ds(t0, BLK), k:k + 1] == eio).astype(jnp.float32)
            off_elem = jnp.sum(ohc * offs, axis=1, keepdims=True)
            dest_ref[pl.ds(t0, BLK), k:k + 1] = (
                rank_scr[pl.ds(t0, BLK), k:k + 1] + off_elem).astype(jnp.int32)

    starts = (jax.lax.broadcasted_iota(jnp.int32, (1, nblk), 1)
              .astype(jnp.float32) * float(BLK))
    bm = jnp.zeros((1, nblk), jnp.float32)
    for e in range(E):
        bm = bm + (starts >= ends[0:1, e:e + 1]).astype(jnp.float32)
    bmap_ref[...] = jnp.minimum(bm, float(E - 1)).astype(jnp.int32)


def _expert_k(bm_ref, xs_ref, w1_ref, b1_ref, w2_ref, b2_ref, y_ref):
    y = jnp.maximum(
        jnp.dot(xs_ref[...], w1_ref[0], preferred_element_type=jnp.float32)
        + b1_ref[0], 0.0)
    y_ref[...] = jnp.dot(y, w2_ref[0],
                         preferred_element_type=jnp.float32) + b2_ref[0]


def _sc_dispatch(m_hbm, destf_hbm, xs_hbm, rows_v, idx_buf, sem, *, n_tok):
    wid = lax.axis_index("s") * 2 + lax.axis_index("c")

    def chunk(c, carry):
        t0 = wid * 128 + c * 64
        pltpu.sync_copy(m_hbm.at[pl.ds(t0, 64)], rows_v)
        for k in range(2):
            pltpu.sync_copy(destf_hbm.at[pl.ds(k * n_tok + t0, 64)], idx_buf)
            pltpu.async_copy(rows_v, xs_hbm.at[idx_buf], sem).wait()
        return carry

    lax.fori_loop(0, 2, chunk, 0)


def _sc_gather2(y_hbm, destf_hbm, y1_hbm, y2_hbm,
                idx1_v, idx2_v, y1_v, y2_v, sem, *, n_tok):
    wid = lax.axis_index("s") * 2 + lax.axis_index("c")

    def chunk(c, carry):
        t0 = wid * 128 + c * 32
        pltpu.sync_copy(destf_hbm.at[pl.ds(t0, 32)], idx1_v)
        pltpu.sync_copy(destf_hbm.at[pl.ds(n_tok + t0, 32)], idx2_v)
        pltpu.async_copy(y_hbm.at[idx1_v], y1_v, sem).wait()
        pltpu.async_copy(y_hbm.at[idx2_v], y2_v, sem).wait()
        pltpu.sync_copy(y1_v, y1_hbm.at[pl.ds(t0, 32)])
        pltpu.sync_copy(y2_v, y2_hbm.at[pl.ds(t0, 32)])
        return carry

    lax.fori_loop(0, 4, chunk, 0)


def _fma_k(h_ref, y1_ref, y2_ref, g_ref, out_ref):
    g1 = g_ref[:, 0:1]
    g2 = g_ref[:, 1:2]
    out_ref[...] = h_ref[...] + g1 * y1_ref[...] + g2 * y2_ref[...]


NSLOTS = 2 * 4096 + E * BLK
NBLK = NSLOTS // BLK

def _sc_mesh():
    return plsc.VectorSubcoreMesh(core_axis_name="c", subcore_axis_name="s")


def _dispatch_call(m, destf):
    n = m.shape[0]
    fn = pl.kernel(
        functools.partial(_sc_dispatch, n_tok=n),
        mesh=_sc_mesh(),
        out_type=jax.ShapeDtypeStruct((NSLOTS, D), jnp.float32),
        scratch_types=[pltpu.VMEM((64, D), jnp.float32),
                       pltpu.VMEM((64,), jnp.int32),
                       pltpu.SemaphoreType.DMA],
    )
    return fn(m, destf)


def _combine_call(h, ys, destf, gates):
    n = h.shape[0]
    fn = pl.kernel(
        functools.partial(_sc_gather2, n_tok=n),
        mesh=_sc_mesh(),
        out_type=[jax.ShapeDtypeStruct((n, D), jnp.float32),
                  jax.ShapeDtypeStruct((n, D), jnp.float32)],
        scratch_types=[pltpu.VMEM((32,), jnp.int32),
                       pltpu.VMEM((32,), jnp.int32),
                       pltpu.VMEM((32, D), jnp.float32),
                       pltpu.VMEM((32, D), jnp.float32),
                       pltpu.SemaphoreType.DMA],
    )
    y1d, y2d = fn(ys, destf)

    nblk = n // TB
    tok_spec = pl.BlockSpec((TB, D), lambda i: (i, 0))
    return pl.pallas_call(
        _fma_k,
        grid=(nblk,),
        in_specs=[tok_spec, tok_spec, tok_spec,
                  pl.BlockSpec((TB, 2), lambda i: (i, 0))],
        out_specs=tok_spec,
        out_shape=jax.ShapeDtypeStruct((n, D), jnp.float32),
    )(h, y1d, y2d, gates)


def _head_k(h_ref, pw_ref, hw_ref, hb_ref, loss_ref, rul_ref, tloss_ref, *, bsz, seq):
    for b in range(bsz):
        hb = h_ref[b * seq:(b + 1) * seq, :]
        sc = jnp.dot(hb, pw_ref[...], preferred_element_type=jnp.float32)
        sc = sc - jnp.max(sc, axis=0, keepdims=True)
        al = jnp.exp(sc)
        al = al / jnp.sum(al, axis=0, keepdims=True)
        pooled = jnp.sum(al * hb, axis=0, keepdims=True)
        rul_ref[b:b + 1, :] = jnp.dot(
            pooled, hw_ref[...], preferred_element_type=jnp.float32) + hb_ref[...]
    tloss_ref[...] = jnp.sum(loss_ref[...], axis=(0, 1), keepdims=True)


def kernel(x, Wp, bp, ln1_g, ln1_b, ln2_g, ln2_b, Wq, bq, Wk, bk, Wv, bv,
           Wo, bo, gW, gb, W1, b1, W2, b2, pool_w, head_W, head_b):
    B, S, _ = x.shape
    N = B * S
    nblk = N // TB
    f32 = jnp.float32

    h = pl.pallas_call(
        _embed_k,
        out_shape=jax.ShapeDtypeStruct((N, D), f32),
    )(x.reshape(N, 1), Wp, bp.reshape(1, D))

    tok_spec = pl.BlockSpec((TB, D), lambda i: (i, 0))
    row_spec = pl.BlockSpec((1, D), lambda i: (0, 0))
    full2 = lambda shape: pl.BlockSpec(shape, lambda i: (0,) * len(shape))
    full0 = lambda shape: pl.BlockSpec(shape, lambda: (0,) * len(shape))

    losses = []
    for l in range(L):
        q, k, v = pl.pallas_call(
            _qkv_k,
            grid=(nblk,),
            in_specs=[tok_spec, row_spec, row_spec,
                      full2((D, D)), row_spec,
                      full2((D, D)), row_spec,
                      full2((D, D)), row_spec],
            out_specs=[tok_spec, tok_spec, tok_spec],
            out_shape=[jax.ShapeDtypeStruct((N, D), f32)] * 3,
        )(h, ln1_g[l].reshape(1, D), ln1_b[l].reshape(1, D),
          Wq[l], bq[l].reshape(1, D), Wk[l], bk[l].reshape(1, D),
          Wv[l], bv[l].reshape(1, D))

        head_spec = pl.BlockSpec((S, 2 * HD), lambda bb, hh: (bb, hh))
        o = pl.pallas_call(
            _attn_k,
            grid=(B, NH // 2),
            in_specs=[head_spec] * 3,
            out_specs=head_spec,
            out_shape=jax.ShapeDtypeStruct((N, D), f32),
        )(q, k, v)

        h = pl.pallas_call(
            _oproj_k,
            grid=(nblk,),
            in_specs=[tok_spec, tok_spec, full2((D, D)), row_spec],
            out_specs=tok_spec,
            out_shape=jax.ShapeDtypeStruct((N, D), f32),
        )(h, o, Wo[l], bo[l].reshape(1, D))

        m, idxp, gates, _, _, lloss = pl.pallas_call(
            functools.partial(_route_k, nblocks=nblk),
            grid=(nblk,),
            in_specs=[tok_spec, row_spec, row_spec,
                      full2((D, E)), pl.BlockSpec((1, E), lambda i: (0, 0))],
            out_specs=[tok_spec,
                       pl.BlockSpec((TB, 2), lambda i: (i, 0)),
                       pl.BlockSpec((TB, 2), lambda i: (i, 0)),
                       pl.BlockSpec((1, E), lambda i: (0, 0)),
                       pl.BlockSpec((1, E), lambda i: (0, 0)),
                       pl.BlockSpec((1, 1), lambda i: (0, 0))],
            out_shape=[jax.ShapeDtypeStruct((N, D), f32),
                       jax.ShapeDtypeStruct((N, 2), jnp.int32),
                       jax.ShapeDtypeStruct((N, 2), f32),
                       jax.ShapeDtypeStruct((1, E), f32),
                       jax.ShapeDtypeStruct((1, E), f32),
                       jax.ShapeDtypeStruct((1, 1), f32)],
        )(h, ln2_g[l].reshape(1, D), ln2_b[l].reshape(1, D),
          gW[l], gb[l].reshape(1, E))
        losses.append(lloss)

        dest, bmap = pl.pallas_call(
            functools.partial(_rank_k, n_tok=N, nblk=NBLK),
            in_specs=[full0((N, 2))],
            out_specs=[full0((N, 2)),
                       pl.BlockSpec((1, NBLK), lambda: (0, 0))],
            out_shape=[jax.ShapeDtypeStruct((N, 2), jnp.int32),
                       jax.ShapeDtypeStruct((1, NBLK), jnp.int32)],
            scratch_shapes=[pltpu.VMEM((N, 2), f32)],
        )(idxp)

        destf = dest.T.reshape(2 * N)
        gatef = gates.T.reshape(2 * N)
        xs = _dispatch_call(m, destf)

        ys = pl.pallas_call(
            _expert_k,
            grid_spec=pltpu.PrefetchScalarGridSpec(
                num_scalar_prefetch=1,
                grid=(NBLK,),
                in_specs=[pl.BlockSpec((BLK, D), lambda b, bm: (b, 0)),
                          pl.BlockSpec((1, D, F), lambda b, bm: (bm[0, b], 0, 0)),
                          pl.BlockSpec((1, 1, F), lambda b, bm: (bm[0, b], 0, 0)),
                          pl.BlockSpec((1, F, D), lambda b, bm: (bm[0, b], 0, 0)),
                          pl.BlockSpec((1, 1, D), lambda b, bm: (bm[0, b], 0, 0))],
                out_specs=pl.BlockSpec((BLK, D), lambda b, bm: (b, 0)),
            ),
            out_shape=jax.ShapeDtypeStruct((NSLOTS, D), f32),
        )(bmap, xs, W1[l], b1[l].reshape(E, 1, F), W2[l], b2[l].reshape(E, 1, D))

        h = _combine_call(h, ys, destf, gates)

    rul, tloss = pl.pallas_call(
        functools.partial(_head_k, bsz=B, seq=S),
        in_specs=[full0((N, D)), full0((D, 1)), full0((D, 1)),
                  pl.BlockSpec((1, 1), lambda: (0, 0)),
                  pl.BlockSpec((L, 1), lambda: (0, 0))],
        out_specs=[pl.BlockSpec((B, 1), lambda: (0, 0)),
                   pl.BlockSpec((1, 1), lambda: (0, 0))],
        out_shape=[jax.ShapeDtypeStruct((B, 1), f32),
                   jax.ShapeDtypeStruct((1, 1), f32)],
    )(h, pool_w, head_W, head_b.reshape(1, 1),
      jnp.concatenate(losses, axis=0).reshape(L, 1))

    return rul, tloss[0, 0]


# speedup vs baseline: 1.1836x; 1.1836x over previous
import functools

import jax
import jax.numpy as jnp
from jax.experimental import pallas as pl

L = 3
D = 768
E = 8
F = 768
NH = 12
HD = 64
TB = 512


def _ln(h, g, b):
    mu = jnp.mean(h, axis=-1, keepdims=True)
    var = jnp.mean((h - mu) ** 2, axis=-1, keepdims=True)
    return (h - mu) / jnp.sqrt(var + 1e-5) * g + b


def _embed_k(x_ref, wp_ref, bp_ref, o_ref):
    o_ref[...] = x_ref[...] * wp_ref[...] + bp_ref[...]


def _qkv_k(h_ref, g_ref, b_ref, wq_ref, bq_ref, wk_ref, bk_ref, wv_ref,
           bv_ref, q_ref, k_ref, v_ref):
    a = _ln(h_ref[...], g_ref[...], b_ref[...])
    q_ref[...] = jnp.dot(a, wq_ref[...], preferred_element_type=jnp.float32) + bq_ref[...]
    k_ref[...] = jnp.dot(a, wk_ref[...], preferred_element_type=jnp.float32) + bk_ref[...]
    v_ref[...] = jnp.dot(a, wv_ref[...], preferred_element_type=jnp.float32) + bv_ref[...]


def _attn_k(q_ref, k_ref, v_ref, o_ref):
    for hh in range(2):
        q = q_ref[:, hh * HD:(hh + 1) * HD] * (1.0 / 8.0)
        k = k_ref[:, hh * HD:(hh + 1) * HD]
        v = v_ref[:, hh * HD:(hh + 1) * HD]
        s = jax.lax.dot_general(q, k, (((1,), (1,)), ((), ())),
                                preferred_element_type=jnp.float32)
        p = jnp.exp(s)
        o = jnp.dot(p, v, preferred_element_type=jnp.float32)
        o_ref[:, hh * HD:(hh + 1) * HD] = o / jnp.sum(p, axis=-1, keepdims=True)


def _oproj_k(h_ref, o_ref, wo_ref, bo_ref, out_ref):
    out_ref[...] = h_ref[...] + jnp.dot(
        o_ref[...], wo_ref[...], preferred_element_type=jnp.float32) + bo_ref[...]


def _moe_k(h_ref, g_ref, b_ref, gw_ref, gb_ref, w1_ref, b1_ref, w2_ref,
           b2_ref, out_ref, imp_ref, load_ref, loss_ref, *, nblocks):
    i = pl.program_id(0)
    h = h_ref[...]
    m = _ln(h, g_ref[...], b_ref[...])

    logits = jnp.dot(m, gw_ref[...], preferred_element_type=jnp.float32) + gb_ref[...]
    logits = logits - jnp.max(logits, axis=-1, keepdims=True)
    ex = jnp.exp(logits)
    probs = ex / jnp.sum(ex, axis=-1, keepdims=True)

    eio = jax.lax.broadcasted_iota(jnp.int32, probs.shape, 1)
    m1 = jnp.max(probs, axis=-1, keepdims=True)
    idx1 = jnp.min(jnp.where(probs == m1, eio, E), axis=-1, keepdims=True)
    masked = jnp.where(eio == idx1, -1.0, probs)
    m2 = jnp.max(masked, axis=-1, keepdims=True)
    idx2 = jnp.min(jnp.where(masked == m2, eio, E), axis=-1, keepdims=True)
    gsum = m1 + m2
    combine = (jnp.where(eio == idx1, m1, 0.0) +
               jnp.where(eio == idx2, m2, 0.0)) / gsum

    acc = jnp.zeros(h.shape, dtype=jnp.float32)
    for e in range(E):
        y = jnp.maximum(
            jnp.dot(m, w1_ref[e], preferred_element_type=jnp.float32)
            + b1_ref[e], 0.0)
        ye = jnp.dot(y, w2_ref[e], preferred_element_type=jnp.float32) + b2_ref[e]
        acc = acc + combine[:, e:e + 1] * ye
    out_ref[...] = h + acc

    imp_blk = jnp.sum(probs, axis=0, keepdims=True)
    load_blk = (jnp.sum(jnp.where(eio == idx1, 1.0, 0.0), axis=0, keepdims=True)
                + jnp.sum(jnp.where(eio == idx2, 1.0, 0.0), axis=0, keepdims=True))

    @pl.when(i == 0)
    def _():
        imp_ref[...] = jnp.zeros_like(imp_ref)
        load_ref[...] = jnp.zeros_like(load_ref)

    imp_ref[...] += imp_blk
    load_ref[...] += load_blk

    @pl.when(i == nblocks - 1)
    def _():
        n_tok = nblocks * h.shape[0]
        loss_ref[...] = ((E / (n_tok * n_tok)) *
                         jnp.sum(imp_ref[...] * load_ref[...],
                                 axis=(0, 1), keepdims=True))


def _head_k(h_ref, pw_ref, hw_ref, hb_ref, loss_ref, rul_ref, tloss_ref, *, bsz, seq):
    for b in range(bsz):
        hb = h_ref[b * seq:(b + 1) * seq, :]
        sc = jnp.dot(hb, pw_ref[...], preferred_element_type=jnp.float32)
        sc = sc - jnp.max(sc, axis=0, keepdims=True)
        al = jnp.exp(sc)
        al = al / jnp.sum(al, axis=0, keepdims=True)
        pooled = jnp.sum(al * hb, axis=0, keepdims=True)
        rul_ref[b:b + 1, :] = jnp.dot(
            pooled, hw_ref[...], preferred_element_type=jnp.float32) + hb_ref[...]
    tloss_ref[...] = jnp.sum(loss_ref[...], axis=(0, 1), keepdims=True)


def kernel(x, Wp, bp, ln1_g, ln1_b, ln2_g, ln2_b, Wq, bq, Wk, bk, Wv, bv,
           Wo, bo, gW, gb, W1, b1, W2, b2, pool_w, head_W, head_b):
    B, S, _ = x.shape
    N = B * S
    nblk = N // TB
    f32 = jnp.float32

    h = pl.pallas_call(
        _embed_k,
        out_shape=jax.ShapeDtypeStruct((N, D), f32),
    )(x.reshape(N, 1), Wp, bp.reshape(1, D))

    tok_spec = pl.BlockSpec((TB, D), lambda i: (i, 0))
    row_spec = pl.BlockSpec((1, D), lambda i: (0, 0))
    full2 = lambda shape: pl.BlockSpec(shape, lambda i: (0,) * len(shape))
    full0 = lambda shape: pl.BlockSpec(shape, lambda: (0,) * len(shape))

    losses = []
    for l in range(L):
        q, k, v = pl.pallas_call(
            _qkv_k,
            grid=(nblk,),
            in_specs=[tok_spec, row_spec, row_spec,
                      full2((D, D)), row_spec,
                      full2((D, D)), row_spec,
                      full2((D, D)), row_spec],
            out_specs=[tok_spec, tok_spec, tok_spec],
            out_shape=[jax.ShapeDtypeStruct((N, D), f32)] * 3,
        )(h, ln1_g[l].reshape(1, D), ln1_b[l].reshape(1, D),
          Wq[l], bq[l].reshape(1, D), Wk[l], bk[l].reshape(1, D),
          Wv[l], bv[l].reshape(1, D))

        head_spec = pl.BlockSpec((S, 2 * HD), lambda bb, hh: (bb, hh))
        o = pl.pallas_call(
            _attn_k,
            grid=(B, NH // 2),
            in_specs=[head_spec] * 3,
            out_specs=head_spec,
            out_shape=jax.ShapeDtypeStruct((N, D), f32),
        )(q, k, v)

        h = pl.pallas_call(
            _oproj_k,
            grid=(nblk,),
            in_specs=[tok_spec, tok_spec, full2((D, D)), row_spec],
            out_specs=tok_spec,
            out_shape=jax.ShapeDtypeStruct((N, D), f32),
        )(h, o, Wo[l], bo[l].reshape(1, D))

        h, _, _, lloss = pl.pallas_call(
            functools.partial(_moe_k, nblocks=nblk),
            grid=(nblk,),
            in_specs=[tok_spec, row_spec, row_spec,
                      full2((D, E)), pl.BlockSpec((1, E), lambda i: (0, 0)),
                      full2((E, D, F)), full2((E, F)),
                      full2((E, F, D)), full2((E, D))],
            out_specs=[tok_spec,
                       pl.BlockSpec((1, E), lambda i: (0, 0)),
                       pl.BlockSpec((1, E), lambda i: (0, 0)),
                       pl.BlockSpec((1, 1), lambda i: (0, 0))],
            out_shape=[jax.ShapeDtypeStruct((N, D), f32),
                       jax.ShapeDtypeStruct((1, E), f32),
                       jax.ShapeDtypeStruct((1, E), f32),
                       jax.ShapeDtypeStruct((1, 1), f32)],
        )(h, ln2_g[l].reshape(1, D), ln2_b[l].reshape(1, D),
          gW[l], gb[l].reshape(1, E), W1[l], b1[l], W2[l], b2[l])
        losses.append(lloss)

    rul, tloss = pl.pallas_call(
        functools.partial(_head_k, bsz=B, seq=S),
        in_specs=[full0((N, D)), full0((D, 1)), full0((D, 1)),
                  pl.BlockSpec((1, 1), lambda: (0, 0)),
                  pl.BlockSpec((L, 1), lambda: (0, 0))],
        out_specs=[pl.BlockSpec((B, 1), lambda: (0, 0)),
                   pl.BlockSpec((1, 1), lambda: (0, 0))],
        out_shape=[jax.ShapeDtypeStruct((B, 1), f32),
                   jax.ShapeDtypeStruct((1, 1), f32)],
    )(h, pool_w, head_W, head_b.reshape(1, 1),
      jnp.concatenate(losses, axis=0).reshape(L, 1))

    return rul, tloss[0, 0]
